# Initial kernel scaffold; baseline (speedup 1.0000x reference)
import jax
import jax.numpy as jnp
from jax.experimental import pallas as pl


def kernel(h, r, t, entity_embedding, relation_embedding):
    raise NotImplementedError("write your pallas kernel here")


# speedup vs baseline: 1.4079x; 1.4079x over previous
import functools

import jax
import jax.numpy as jnp
from jax import lax
from jax.experimental import pallas as pl
from jax.experimental.pallas import tpu as pltpu
from jax.experimental.pallas import tpu_sc as plsc

N_ENTITY = 1000000
N_RELATION = 1000
DIM = 64
GAMMA = 12.0
EMB_RANGE = (GAMMA + 2.0) / DIM
PI = 3.141592653589793
BATCH = 4096
PHASE_K = PI / EMB_RANGE

NC, NS, L = 2, 16, 16
NW = NC * NS
BPW = BATCH // NW
NCHUNK = DIM // L

_SIN_C = (1.0, -0.1666666567325592, 0.008333321660757065,
          -0.00019840533786918968, 2.753584794845665e-06,
          -2.472880211712436e-08, 1.3613066229822834e-10)
_COS_C = (1.0, -0.5, 0.0416666641831398, -0.0013888863613829017,
          2.480055445630569e-05, -2.7534812829799193e-07,
          2.0603632133742167e-09, -9.722611604701115e-12)


def _horner(coeffs, t):
    acc = jnp.full((L,), coeffs[-1], jnp.float32)
    for c in coeffs[-2::-1]:
        acc = acc * t + jnp.float32(c)
    return acc


_SQRT12_C = (0.26855847239494324, 1.1340605020523071, -0.6584334969520569,
             0.3633367717266083, -0.13294294476509094, 0.027977269142866135,
             -0.0025564369279891253)


def _modulus16(re, im):
    a = jnp.abs(re)
    b = jnp.abs(im)
    hi = jnp.maximum(a, b)
    lo = jnp.minimum(a, b)
    ratio = lo / (hi + jnp.float32(1e-30))
    return hi * _horner(_SQRT12_C, jnp.float32(1.0) + ratio * ratio)


_mesh = plsc.VectorSubcoreMesh(core_axis_name="c", subcore_axis_name="s")


@functools.partial(
    pl.kernel,
    out_type=jax.ShapeDtypeStruct((BATCH,), jnp.float32),
    mesh=_mesh,
    compiler_params=pltpu.CompilerParams(needs_layout_passes=False,
                                         use_tc_tiling_on_sc=False),
    scratch_types=[
        pltpu.VMEM((BPW,), jnp.int32),
        pltpu.VMEM((BPW,), jnp.int32),
        pltpu.VMEM((BPW,), jnp.int32),
        pltpu.VMEM((BPW, 2 * DIM), jnp.float32),
        pltpu.VMEM((BPW, 2 * DIM), jnp.float32),
        pltpu.VMEM((BPW, DIM), jnp.float32),
        pltpu.VMEM((BPW, L), jnp.float32),
        pltpu.VMEM((BPW,), jnp.float32),
        pltpu.SemaphoreType.DMA,
        pltpu.SemaphoreType.DMA,
        pltpu.SemaphoreType.DMA,
    ],
)
def _rotate_body(h_hbm, r_hbm, t_hbm, ent_hbm, rel_hbm, out_hbm,
                 hidx, ridx, tidx, head_v, tail_v, rel_v, part_v, out_v,
                 sem_h, sem_t, sem_r):
    wid = lax.axis_index("s") * NC + lax.axis_index("c")
    base = wid * BPW

    pltpu.sync_copy(h_hbm.at[pl.ds(base, BPW)], hidx)
    pltpu.sync_copy(t_hbm.at[pl.ds(base, BPW)], tidx)
    pltpu.sync_copy(r_hbm.at[pl.ds(base, BPW)], ridx)

    cp_h = pltpu.async_copy(ent_hbm.at[hidx], head_v, sem_h)
    cp_t = pltpu.async_copy(ent_hbm.at[tidx], tail_v, sem_t)
    cp_r = pltpu.async_copy(rel_hbm.at[ridx], rel_v, sem_r)
    cp_r.wait()
    cp_h.wait()
    cp_t.wait()

    def item(i, carry):
        acc = jnp.zeros((L,), jnp.float32)
        for j in range(NCHUNK):
            lo = j * L
            ph = rel_v[i, pl.ds(lo, L)] * jnp.float32(PHASE_K)
            t2 = ph * ph
            cos_r = _horner(_COS_C, t2)
            sin_r = ph * _horner(_SIN_C, t2)
            re_t = tail_v[i, pl.ds(lo, L)]
            im_t = tail_v[i, pl.ds(DIM + lo, L)]
            re_h = head_v[i, pl.ds(lo, L)]
            im_h = head_v[i, pl.ds(DIM + lo, L)]
            re_s = cos_r * re_t + sin_r * im_t - re_h
            im_s = cos_r * im_t - sin_r * re_t - im_h
            acc = acc + _modulus16(re_s, im_s)
        part_v[i, pl.ds(0, L)] = acc
        return carry

    lax.fori_loop(0, BPW, item, 0, unroll=False)

    iota = lax.iota(jnp.int32, L)
    for g in range(BPW // L):
        items = iota + jnp.int32(g * L)
        tot = jnp.zeros((L,), jnp.float32)
        for d in range(L):
            tot = tot + plsc.load_gather(
                part_v, [items, jnp.full((L,), d, jnp.int32)])
        out_v[pl.ds(g * L, L)] = -tot

    pltpu.sync_copy(out_v, out_hbm.at[pl.ds(base, BPW)])


def kernel(h, r, t, entity_embedding, relation_embedding):
    flat = _rotate_body(h.astype(jnp.int32), r.astype(jnp.int32),
                        t.astype(jnp.int32), entity_embedding,
                        relation_embedding)
    return flat.reshape(BATCH, 1)
